# Initial kernel scaffold; baseline (speedup 1.0000x reference)
import jax
import jax.numpy as jnp
from jax.experimental import pallas as pl


def kernel(x, bag_sizes, W_enc, b_enc, V, w_att, W_ins, b_ins, W_bag, b_bag):
    raise NotImplementedError("write your pallas kernel here")


# speedup vs baseline: 4.2869x; 4.2869x over previous
import jax
import jax.numpy as jnp
from jax import lax
from jax.experimental import pallas as pl
from jax.experimental.pallas import tpu as pltpu

N = 32768
D_IN = 256
D_HID = 128
D_ATT = 64
B = 16
BLK = 1024


def _fused_body(starts_ref, ends_ref, x_ref, W_enc_ref, b_enc_ref, V_ref,
                w_att_ref, W_bag_ref, b_bag_ref, out_ref, z_acc, s_acc):
    blk = pl.program_id(0)
    nblk = pl.num_programs(0)

    x = x_ref[...]
    h = jnp.dot(x, W_enc_ref[...], preferred_element_type=jnp.float32)
    h = h + b_enc_ref[...]
    t = jnp.tanh(jnp.dot(h, V_ref[...], preferred_element_type=jnp.float32))
    a = jnp.dot(t, w_att_ref[...], preferred_element_type=jnp.float32)
    e = jnp.exp(a)

    i = blk * BLK + lax.broadcasted_iota(jnp.int32, (BLK, 1), 0)
    m = (i >= starts_ref[...]) & (i < ends_ref[...])
    me = jnp.where(m, e, 0.0)

    zp = lax.dot_general(me, h, (((0,), (0,)), ((), ())),
                         preferred_element_type=jnp.float32)
    ones = jnp.ones((BLK, 1), jnp.float32)
    sp = lax.dot_general(me, ones, (((0,), (0,)), ((), ())),
                         preferred_element_type=jnp.float32)

    @pl.when(blk == 0)
    def _():
        z_acc[...] = zp
        s_acc[...] = sp

    @pl.when(blk > 0)
    def _():
        z_acc[...] += zp
        s_acc[...] += sp

    @pl.when(blk == nblk - 1)
    def _():
        z = z_acc[...]
        s = s_acc[...]
        num = lax.dot_general(z, W_bag_ref[...], (((1,), (0,)), ((), ())),
                              preferred_element_type=jnp.float32)
        denom = jnp.where(s > 0, s, 1.0)
        out_ref[...] = num / denom + b_bag_ref[...]


def kernel(x, bag_sizes, W_enc, b_enc, V, w_att, W_ins, b_ins, W_bag, b_bag):
    starts = bag_sizes[:B].reshape(1, B)
    ends = bag_sizes[1:].reshape(1, B)
    nblk = N // BLK
    out = pl.pallas_call(
        _fused_body,
        grid=(nblk,),
        in_specs=[
            pl.BlockSpec((1, B), lambda i: (0, 0)),
            pl.BlockSpec((1, B), lambda i: (0, 0)),
            pl.BlockSpec((BLK, D_IN), lambda i: (i, 0)),
            pl.BlockSpec((D_IN, D_HID), lambda i: (0, 0)),
            pl.BlockSpec((1, D_HID), lambda i: (0, 0)),
            pl.BlockSpec((D_HID, D_ATT), lambda i: (0, 0)),
            pl.BlockSpec((D_ATT, 1), lambda i: (0, 0)),
            pl.BlockSpec((D_HID, 1), lambda i: (0, 0)),
            pl.BlockSpec((1, 1), lambda i: (0, 0)),
        ],
        out_specs=pl.BlockSpec((B, 1), lambda i: (0, 0)),
        out_shape=jax.ShapeDtypeStruct((B, 1), jnp.float32),
        scratch_shapes=[
            pltpu.VMEM((B, D_HID), jnp.float32),
            pltpu.VMEM((B, 1), jnp.float32),
        ],
    )(starts, ends, x, W_enc, b_enc.reshape(1, D_HID), V, w_att,
      W_bag, b_bag.reshape(1, 1))
    return out
